# Initial kernel scaffold; baseline (speedup 1.0000x reference)
import jax
import jax.numpy as jnp
from jax.experimental import pallas as pl


def kernel(subword_idx, offsets, table):
    raise NotImplementedError("write your pallas kernel here")


# speedup vs baseline: 27.9057x; 27.9057x over previous
import functools

import jax
import jax.numpy as jnp
from jax import lax
from jax.experimental import pallas as pl
from jax.experimental.pallas import tpu as pltpu
from jax.experimental.pallas import tpu_sc as plsc

NC = 2
NS = 16
NW = NC * NS
L = 16
CHUNK = 512


@functools.lru_cache(maxsize=None)
def _build(T, B, V, D):
    assert D == 64 and B % NW == 0 and CHUNK % 8 == 0
    bags_w = B // NW
    nk = D // L

    mesh = plsc.VectorSubcoreMesh(core_axis_name="c", subcore_axis_name="s")

    def sread(ref, i):
        return ref[pl.ds(i, L)][0]

    @functools.partial(
        pl.kernel,
        mesh=mesh,
        compiler_params=pltpu.CompilerParams(use_tc_tiling_on_sc=False),
        out_type=jax.ShapeDtypeStruct((B, D), jnp.float32),
        scratch_types=[
            pltpu.VMEM((bags_w + 24,), jnp.int32),
            pltpu.VMEM((CHUNK,), jnp.int32),
            pltpu.VMEM((CHUNK, D), jnp.float32),
            pltpu.VMEM((bags_w, D), jnp.float32),
            pltpu.SemaphoreType.DMA,
        ],
    )
    def emb(idx_hbm, offs_hbm, table_hbm, out_hbm, offs_v, idx_v, rows_v, out_v, sem):
        wid = lax.axis_index("s") * NC + lax.axis_index("c")
        bag0 = wid * bags_w
        pltpu.sync_copy(offs_hbm.at[pl.ds(bag0, bags_w + 8)], offs_v.at[pl.ds(0, bags_w + 8)])
        p0 = sread(offs_v, 0)
        p1 = sread(offs_v, bags_w)
        a0 = (p0 // 8) * 8
        nchunks = jnp.maximum((p1 - a0 + CHUNK - 1) // CHUNK, 1)

        zero = jnp.zeros((L,), jnp.float32)

        def chunk_body(c, state):
            b = state[0]
            g0 = a0 + c * CHUNK
            gend = jnp.minimum(g0 + CHUNK, p1)
            pltpu.sync_copy(idx_hbm.at[pl.ds(g0, CHUNK)], idx_v)
            pltpu.async_copy(table_hbm.at[idx_v], rows_v, sem).wait()

            def row_body(r, accs):
                lr = r - g0
                return tuple(
                    accs[k] + rows_v[lr, k * L:(k + 1) * L] for k in range(nk)
                )

            def bs_body(_, lohi):
                lo, hi = lohi
                mid = (lo + hi + 1) // 2
                take = sread(offs_v, mid) <= gend
                return (jnp.where(take, mid, lo), jnp.where(take, hi, mid - 1))

            b_end, _ = lax.fori_loop(0, 10, bs_body, (b, jnp.int32(bags_w)))

            def bag_body(b, accs):
                s = sread(offs_v, b)
                e = sread(offs_v, b + 1)
                lo = jnp.maximum(s, g0)
                accs = lax.fori_loop(lo, e, row_body, accs)
                cntv = jnp.full((L,), jnp.maximum(e - s, 1))
                sc = 1.0 / cntv.astype(jnp.float32)
                for k in range(nk):
                    out_v[b, k * L:(k + 1) * L] = accs[k] * sc
                return (zero,) * nk

            st = (b_end,) + lax.fori_loop(b, b_end, bag_body, state[1:])
            bc = jnp.minimum(st[0], bags_w)
            lo = jnp.minimum(jnp.maximum(sread(offs_v, bc), g0), gend)
            accs = lax.fori_loop(lo, gend, row_body, st[1:])
            return (st[0],) + accs

        lax.fori_loop(0, nchunks, chunk_body, (jnp.int32(0),) + (zero,) * nk)
        pltpu.sync_copy(out_v, out_hbm.at[pl.ds(bag0, bags_w)])

    return emb


def kernel(subword_idx, offsets, table):
    T = subword_idx.shape[0]
    B = offsets.shape[0]
    V, D = table.shape
    emb = _build(T, B, V, D)
    idx_p = jnp.concatenate([subword_idx, jnp.zeros((CHUNK,), jnp.int32)])
    offs_p = jnp.concatenate([offsets, jnp.full((8,), T, jnp.int32)])
    return emb(idx_p, offs_p, table)
